# baseline (speedup 1.0000x reference)
import jax
import jax.numpy as jnp
from jax.experimental import pallas as pl
from jax.experimental.pallas import tpu as pltpu

N = 2048
S = 64
D = 128
H = 4
DH = D // H
RB = 512
NB = N // RB
GP = RB // (N // S)

_NEG = -9e15


def _masked_gat_rows(e_src_blk, e_dst_t, e_dst_max, colmean, adj_blk, hs):
    outs = []
    for h in range(H):
        es = e_src_blk[:, h:h + 1]
        lg = es + e_dst_t[H + h:H + h + 1, :]
        lg = jnp.maximum(lg, 0.2 * lg)
        lg = jnp.where(adj_blk > 0, lg, _NEG)
        t = es + e_dst_max[0:1, H + h:H + h + 1]
        m = jnp.maximum(t, 0.2 * t)
        p = jnp.exp(lg - m)
        s = jnp.sum(p, axis=1, keepdims=True)
        o = jnp.dot(p, hs[:, h * DH:(h + 1) * DH],
                    preferred_element_type=jnp.float32)
        outs.append(jnp.where(s > 0, o / s,
                              colmean[0:1, h * DH:(h + 1) * DH]))
    ob = jnp.concatenate(outs, axis=1)
    return jnp.where(ob > 0, ob, jnp.exp(jnp.minimum(ob, 0.0)) - 1.0)


def _fused_kernel(x_ref, linW_ref, linb_ref, wn_ref, acat_n_ref,
                  adj_ref, adjs_ref, ws_ref, acat_s_ref, pool_ref,
                  out_ref, hs_scr, es_scr, eE_scr, eF_scr, ea_scr, eb_scr,
                  vaug_scr, cmean_scr, obuf_scr, pooled_scr,
                  abuf_scr, dma_sem):
    i = pl.program_id(0)

    def _adj_copy(blk, slot):
        return pltpu.make_async_copy(
            adj_ref.at[pl.ds(blk * RB, RB), :], abuf_scr.at[slot],
            dma_sem.at[slot])

    @pl.when(i == 0)
    def _start_first():
        _adj_copy(0, 0).start()

    @pl.when(i + 1 < NB)
    def _prefetch_next():
        _adj_copy(i + 1, (i + 1) % 2).start()

    @pl.when(i == 0)
    def _init():
        h0 = jnp.dot(x_ref[:], linW_ref[:],
                     preferred_element_type=jnp.float32) + linb_ref[:]
        hs = jnp.dot(h0, wn_ref[:], preferred_element_type=jnp.float32)
        hs_scr[:] = hs
        es = jnp.dot(hs, acat_n_ref[:], preferred_element_type=jnp.float32)
        es_scr[:] = es
        edt = jax.lax.dot_general(
            acat_n_ref[:], hs, (((0,), (1,)), ((), ())),
            preferred_element_type=jnp.float32)
        edmax = jnp.max(es, axis=0, keepdims=True)
        cmean = jnp.mean(hs, axis=0, keepdims=True)
        for h in range(H):
            esh = es[:, h:h + 1]
            t0 = esh + edmax[0:1, H + h:H + h + 1]
            m = jnp.maximum(t0, 0.2 * t0)
            ea_scr[h] = jnp.exp(esh - m).astype(jnp.bfloat16)
            eb_scr[h] = jnp.exp(0.2 * esh - m).astype(jnp.bfloat16)
            eE_scr[h] = jnp.exp(edt[H + h:H + h + 1, :]).astype(jnp.bfloat16)
            eF_scr[h] = jnp.exp(0.2 * edt[H + h:H + h + 1, :]
                                ).astype(jnp.bfloat16)
            cmean_scr[h] = cmean[0:1, h * DH:(h + 1) * DH]
            vaug_scr[h, :, 0:DH] = (
                hs[:, h * DH:(h + 1) * DH].astype(jnp.bfloat16))
            vaug_scr[h, :, DH:DH + 1] = jnp.ones((N, 1), jnp.bfloat16)
            vaug_scr[h, :, DH + 1:] = jnp.zeros((N, DH - 1), jnp.bfloat16)

    _adj_copy(i, i % 2).wait()
    adj_blk = abuf_scr[i % 2].astype(jnp.bfloat16)

    def _head(h, carry):
        ea = ea_scr[h, pl.ds(i * RB, RB), :]
        eb = eb_scr[h, pl.ds(i * RB, RB), :]
        p = jnp.maximum(ea * eE_scr[h][:, :], eb * eF_scr[h][:, :]) * adj_blk
        r = jnp.dot(p, vaug_scr[h], preferred_element_type=jnp.float32)
        o = r[:, :DH]
        s = r[:, DH:DH + 1]
        obuf_scr[h] = jnp.where(s > 0, o / s, cmean_scr[h][:, :])
        return carry

    jax.lax.fori_loop(0, H, _head, 0)
    ob = jnp.concatenate([obuf_scr[h] for h in range(H)], axis=1)
    ob = jnp.where(ob > 0, ob, jnp.exp(jnp.minimum(ob, 0.0)) - 1.0)
    pooled_scr[pl.ds(i * GP, GP), :] = jnp.dot(
        pool_ref[:], ob, preferred_element_type=jnp.float32)

    @pl.when(i == NB - 1)
    def _final():
        hs2 = jnp.dot(pooled_scr[:], ws_ref[:],
                      preferred_element_type=jnp.float32)
        es2 = jnp.dot(hs2, acat_s_ref[:], preferred_element_type=jnp.float32)
        edt2 = jax.lax.dot_general(
            acat_s_ref[:], hs2, (((0,), (1,)), ((), ())),
            preferred_element_type=jnp.float32)
        edmax2 = jnp.max(es2, axis=0, keepdims=True)
        cmean2 = jnp.mean(hs2, axis=0, keepdims=True)
        out_ref[:] = _masked_gat_rows(es2, edt2, edmax2, cmean2,
                                      adjs_ref[:], hs2)


def _flat_w(W):
    return jnp.transpose(W, (1, 0, 2)).reshape(D, H * DH)


def _acat(a_src, a_dst):
    eye = jnp.repeat(jnp.eye(H, dtype=jnp.float32), DH, axis=0)
    return jnp.concatenate([eye * a_src.reshape(-1)[:, None],
                            eye * a_dst.reshape(-1)[:, None]], axis=1)


def kernel(x, adj_node, adj_substation, lin_W, lin_b,
           node0_W, node0_a_src, node0_a_dst,
           node1_W, node1_a_src, node1_a_dst,
           sub0_W, sub0_a_src, sub0_a_dst,
           sub1_W, sub1_a_src, sub1_a_dst):
    del node0_W, node0_a_src, node0_a_dst, sub0_W, sub0_a_src, sub0_a_dst

    wn = _flat_w(node1_W)
    acat_n = _acat(node1_a_src, node1_a_dst)
    ws = _flat_w(sub1_W)
    acat_s = _acat(sub1_a_src, sub1_a_dst)
    linb2 = lin_b.reshape(1, D)
    pool_mat = jnp.kron(jnp.eye(GP, dtype=jnp.float32),
                        jnp.full((1, N // S), 1.0 / (N // S), jnp.float32))

    return pl.pallas_call(
        _fused_kernel,
        grid=(NB,),
        in_specs=[
            pl.BlockSpec((N, D), lambda i: (0, 0)),
            pl.BlockSpec((D, D), lambda i: (0, 0)),
            pl.BlockSpec((1, D), lambda i: (0, 0)),
            pl.BlockSpec((D, D), lambda i: (0, 0)),
            pl.BlockSpec((D, 2 * H), lambda i: (0, 0)),
            pl.BlockSpec(memory_space=pltpu.MemorySpace.HBM),
            pl.BlockSpec((S, S), lambda i: (0, 0)),
            pl.BlockSpec((D, D), lambda i: (0, 0)),
            pl.BlockSpec((D, 2 * H), lambda i: (0, 0)),
            pl.BlockSpec((GP, RB), lambda i: (0, 0)),
        ],
        out_specs=pl.BlockSpec((S, D), lambda i: (0, 0)),
        out_shape=jax.ShapeDtypeStruct((S, D), jnp.float32),
        scratch_shapes=[
            pltpu.VMEM((N, D), jnp.float32),
            pltpu.VMEM((N, 2 * H), jnp.float32),
            pltpu.VMEM((H, 1, N), jnp.bfloat16),
            pltpu.VMEM((H, 1, N), jnp.bfloat16),
            pltpu.VMEM((H, N, 1), jnp.bfloat16),
            pltpu.VMEM((H, N, 1), jnp.bfloat16),
            pltpu.VMEM((H, N, 2 * DH), jnp.bfloat16),
            pltpu.VMEM((H, 1, DH), jnp.float32),
            pltpu.VMEM((H, RB, DH), jnp.float32),
            pltpu.VMEM((S, D), jnp.float32),
            pltpu.VMEM((2, RB, N), jnp.float32),
            pltpu.SemaphoreType.DMA((2,)),
        ],
    )(x, lin_W, linb2, wn, acat_n, adj_node, adj_substation, ws, acat_s,
      pool_mat)


# speedup vs baseline: 1.2244x; 1.2244x over previous
import jax
import jax.numpy as jnp
from jax.experimental import pallas as pl
from jax.experimental.pallas import tpu as pltpu

N = 2048
S = 64
D = 128
H = 4
DH = D // H
RB = 512
NB = N // RB
GP = RB // (N // S)

_NEG = -9e15


def _masked_gat_rows(e_src_blk, e_dst_t, e_dst_max, colmean, adj_blk, hs):
    outs = []
    for h in range(H):
        es = e_src_blk[:, h:h + 1]
        lg = es + e_dst_t[H + h:H + h + 1, :]
        lg = jnp.maximum(lg, 0.2 * lg)
        lg = jnp.where(adj_blk > 0, lg, _NEG)
        t = es + e_dst_max[0:1, H + h:H + h + 1]
        m = jnp.maximum(t, 0.2 * t)
        p = jnp.exp(lg - m)
        s = jnp.sum(p, axis=1, keepdims=True)
        o = jnp.dot(p, hs[:, h * DH:(h + 1) * DH],
                    preferred_element_type=jnp.float32)
        outs.append(jnp.where(s > 0, o / s,
                              colmean[0:1, h * DH:(h + 1) * DH]))
    ob = jnp.concatenate(outs, axis=1)
    return jnp.where(ob > 0, ob, jnp.exp(jnp.minimum(ob, 0.0)) - 1.0)


def _fused_kernel(x_ref, linW_ref, linb_ref, wn_ref, acat_n_ref,
                  adj_ref, adjs_ref, ws_ref, acat_s_ref, pool_ref,
                  out_ref, hs_scr, es_scr, edt_scr, edt02_scr, vaug_scr,
                  edmax_scr, cmean_scr, pooled_scr, abuf_scr, dma_sem):
    i = pl.program_id(0)

    def _adj_copy(blk, slot):
        return pltpu.make_async_copy(
            adj_ref.at[pl.ds(blk * RB, RB), :], abuf_scr.at[slot],
            dma_sem.at[slot])

    @pl.when(i == 0)
    def _start_first():
        _adj_copy(0, 0).start()

    @pl.when(i + 1 < NB)
    def _prefetch_next():
        _adj_copy(i + 1, (i + 1) % 2).start()

    @pl.when(i == 0)
    def _init():
        h0 = jnp.dot(x_ref[:], linW_ref[:],
                     preferred_element_type=jnp.float32) + linb_ref[:]
        hs = jnp.dot(h0, wn_ref[:], preferred_element_type=jnp.float32)
        hs_scr[:] = hs
        es = jnp.dot(hs, acat_n_ref[:], preferred_element_type=jnp.float32)
        es_scr[:] = es
        edt = jax.lax.dot_general(
            acat_n_ref[:], hs, (((0,), (1,)), ((), ())),
            preferred_element_type=jnp.float32)
        edt_scr[:] = jnp.exp(edt).astype(jnp.bfloat16)
        edt02_scr[:] = jnp.exp(0.2 * edt).astype(jnp.bfloat16)
        edmax_scr[:] = jnp.max(es, axis=0, keepdims=True)
        cmean_scr[:] = jnp.mean(hs, axis=0, keepdims=True)
        for h in range(H):
            vaug_scr[:, h * 2 * DH:h * 2 * DH + DH] = (
                hs[:, h * DH:(h + 1) * DH].astype(jnp.bfloat16))
            vaug_scr[:, h * 2 * DH + DH:h * 2 * DH + DH + 1] = (
                jnp.ones((N, 1), jnp.bfloat16))
            vaug_scr[:, h * 2 * DH + DH + 1:(h + 1) * 2 * DH] = (
                jnp.zeros((N, DH - 1), jnp.bfloat16))

    _adj_copy(i, i % 2).wait()
    es_blk = es_scr[pl.ds(i * RB, RB), :]
    adj_blk = abuf_scr[i % 2].astype(jnp.bfloat16)
    cmean = cmean_scr[:]
    outs = []
    for h in range(H):
        es = es_blk[:, h:h + 1]
        t0 = es + edmax_scr[0:1, H + h:H + h + 1]
        m = jnp.maximum(t0, 0.2 * t0)
        ea = jnp.exp(es - m).astype(jnp.bfloat16)
        eb = jnp.exp(0.2 * es - m).astype(jnp.bfloat16)
        p = jnp.maximum(ea * edt_scr[H + h:H + h + 1, :],
                        eb * edt02_scr[H + h:H + h + 1, :]) * adj_blk
        r = jnp.dot(p, vaug_scr[:, h * 2 * DH:(h + 1) * 2 * DH],
                    preferred_element_type=jnp.float32)
        o = r[:, :DH]
        s = r[:, DH:DH + 1]
        outs.append(jnp.where(s > 0, o / s,
                              cmean[0:1, h * DH:(h + 1) * DH]))
    ob = jnp.concatenate(outs, axis=1)
    ob = jnp.where(ob > 0, ob, jnp.exp(jnp.minimum(ob, 0.0)) - 1.0)
    pooled_scr[pl.ds(i * GP, GP), :] = jnp.dot(
        pool_ref[:], ob, preferred_element_type=jnp.float32)

    @pl.when(i == NB - 1)
    def _final():
        hs2 = jnp.dot(pooled_scr[:], ws_ref[:],
                      preferred_element_type=jnp.float32)
        es2 = jnp.dot(hs2, acat_s_ref[:], preferred_element_type=jnp.float32)
        edt2 = jax.lax.dot_general(
            acat_s_ref[:], hs2, (((0,), (1,)), ((), ())),
            preferred_element_type=jnp.float32)
        edmax2 = jnp.max(es2, axis=0, keepdims=True)
        cmean2 = jnp.mean(hs2, axis=0, keepdims=True)
        out_ref[:] = _masked_gat_rows(es2, edt2, edmax2, cmean2,
                                      adjs_ref[:], hs2)


def _flat_w(W):
    return jnp.transpose(W, (1, 0, 2)).reshape(D, H * DH)


def _acat(a_src, a_dst):
    eye = jnp.repeat(jnp.eye(H, dtype=jnp.float32), DH, axis=0)
    return jnp.concatenate([eye * a_src.reshape(-1)[:, None],
                            eye * a_dst.reshape(-1)[:, None]], axis=1)


def kernel(x, adj_node, adj_substation, lin_W, lin_b,
           node0_W, node0_a_src, node0_a_dst,
           node1_W, node1_a_src, node1_a_dst,
           sub0_W, sub0_a_src, sub0_a_dst,
           sub1_W, sub1_a_src, sub1_a_dst):
    del node0_W, node0_a_src, node0_a_dst, sub0_W, sub0_a_src, sub0_a_dst

    wn = _flat_w(node1_W)
    acat_n = _acat(node1_a_src, node1_a_dst)
    ws = _flat_w(sub1_W)
    acat_s = _acat(sub1_a_src, sub1_a_dst)
    linb2 = lin_b.reshape(1, D)
    pool_mat = jnp.kron(jnp.eye(GP, dtype=jnp.float32),
                        jnp.full((1, N // S), 1.0 / (N // S), jnp.float32))

    return pl.pallas_call(
        _fused_kernel,
        grid=(NB,),
        in_specs=[
            pl.BlockSpec((N, D), lambda i: (0, 0)),
            pl.BlockSpec((D, D), lambda i: (0, 0)),
            pl.BlockSpec((1, D), lambda i: (0, 0)),
            pl.BlockSpec((D, D), lambda i: (0, 0)),
            pl.BlockSpec((D, 2 * H), lambda i: (0, 0)),
            pl.BlockSpec(memory_space=pltpu.MemorySpace.HBM),
            pl.BlockSpec((S, S), lambda i: (0, 0)),
            pl.BlockSpec((D, D), lambda i: (0, 0)),
            pl.BlockSpec((D, 2 * H), lambda i: (0, 0)),
            pl.BlockSpec((GP, RB), lambda i: (0, 0)),
        ],
        out_specs=pl.BlockSpec((S, D), lambda i: (0, 0)),
        out_shape=jax.ShapeDtypeStruct((S, D), jnp.float32),
        scratch_shapes=[
            pltpu.VMEM((N, D), jnp.float32),
            pltpu.VMEM((N, 2 * H), jnp.float32),
            pltpu.VMEM((2 * H, N), jnp.bfloat16),
            pltpu.VMEM((2 * H, N), jnp.bfloat16),
            pltpu.VMEM((N, 2 * D), jnp.bfloat16),
            pltpu.VMEM((1, 2 * H), jnp.float32),
            pltpu.VMEM((1, D), jnp.float32),
            pltpu.VMEM((S, D), jnp.float32),
            pltpu.VMEM((2, RB, N), jnp.float32),
            pltpu.SemaphoreType.DMA((2,)),
        ],
    )(x, lin_W, linb2, wn, acat_n, adj_node, adj_substation, ws, acat_s,
      pool_mat)
